# baseline (speedup 1.0000x reference)
import functools

import jax
import jax.numpy as jnp
from jax import lax
from jax.experimental import pallas as pl
from jax.experimental.pallas import tpu as pltpu
from jax.experimental.pallas import tpu_sc as plsc

N = 10000
E = 320000
D = 128
H = 128

NC = 2
NS = 16
LN = 16
NW = NC * NS
E_HALF1 = 115200
RPT = 624
RTAIL = N - RPT * NS

_sc_mesh = functools.partial(
    plsc.VectorSubcoreMesh,
    core_axis_name="c", subcore_axis_name="s", num_cores=NC, num_subcores=NS)


def _tables_body(ns_ref, wp_ref, bp_ref, w1a_ref, w1b_ref, be1_ref,
                 a_ref, b_ref):
    nm = jnp.dot(ns_ref[...], wp_ref[...],
                 preferred_element_type=jnp.float32) + bp_ref[...]
    a_ref[...] = jnp.dot(nm, w1a_ref[...],
                         preferred_element_type=jnp.float32) + be1_ref[...]
    b_ref[...] = jnp.dot(nm, w1b_ref[...],
                         preferred_element_type=jnp.float32)


def _make_tables(ns, wp, bp, w1a, w1b, be1):
    blk = 1000
    grid = N // blk
    row_spec = pl.BlockSpec((blk, D), lambda i: (i, 0))
    full = lambda s: pl.BlockSpec(s, lambda i: (0,) * len(s))
    return pl.pallas_call(
        _tables_body,
        grid=(grid,),
        in_specs=[row_spec, full((D, D)), full((1, D)),
                  full((D, H)), full((D, H)), full((1, H))],
        out_specs=[pl.BlockSpec((blk, H), lambda i: (i, 0))] * 2,
        out_shape=[jax.ShapeDtypeStruct((N, H), jnp.float32)] * 2,
    )(ns, wp, bp, w1a, w1b, be1)


def _gather_body(eoff, ew, CH, a_hbm, b_hbm, src_hbm, dst_hbm, h1_hbm,
                 sidx0, sidx1, didx0, didx1, abuf0, abuf1, bbuf0, bbuf1,
                 gi0, gi1, gg0, gg1, gs0, gs1):
    ncht = ew // CH
    cid = lax.axis_index("c")
    sid = lax.axis_index("s")
    wid = sid * NC + cid
    base_i = eoff + wid * ew
    base_o = wid * ew

    sidx = (sidx0, sidx1)
    didx = (didx0, didx1)
    abuf = (abuf0, abuf1)
    bbuf = (bbuf0, bbuf1)
    gi = (gi0, gi1)
    gg = (gg0, gg1)
    gs = (gs0, gs1)

    def idx_issue(c, p):
        sl = pl.ds(base_i + c * CH, CH)
        pltpu.async_copy(src_hbm.at[sl], sidx[p], gi[p])
        pltpu.async_copy(dst_hbm.at[sl], didx[p], gi[p])

    def idx_wait(p):
        pltpu.make_async_copy(src_hbm.at[pl.ds(0, CH)], sidx[p], gi[p]).wait()
        pltpu.make_async_copy(dst_hbm.at[pl.ds(0, CH)], didx[p], gi[p]).wait()

    def gather_issue(p):
        pltpu.async_copy(a_hbm.at[sidx[p]], abuf[p], gg[p])
        pltpu.async_copy(b_hbm.at[didx[p]], bbuf[p], gg[p])

    def gather_wait(p):
        pltpu.make_async_copy(a_hbm.at[sidx[p]], abuf[p], gg[p]).wait()
        pltpu.make_async_copy(b_hbm.at[didx[p]], bbuf[p], gg[p]).wait()

    def store_wait(p):
        pltpu.make_async_copy(
            abuf[p], h1_hbm.at[pl.ds(base_o, CH)], gs[p]).wait()

    def stage(c, p, do_store_wait=True, do_idx=True, do_gather=True):
        pn = 1 - p
        gather_wait(p)
        if do_idx:
            idx_issue(c + 2, p)
        if do_store_wait:
            store_wait(pn)
        if do_gather:
            idx_wait(pn)
            gather_issue(pn)

        def row(r, c2):
            for j in range(H // LN):
                sl = pl.ds(j * LN, LN)
                abuf[p][r, sl] = abuf[p][r, sl] + bbuf[p][r, sl]
            return c2

        lax.fori_loop(0, CH, row, 0)
        pltpu.async_copy(abuf[p], h1_hbm.at[pl.ds(base_o + c * CH, CH)],
                         gs[p])

    sl0 = pl.ds(base_i, CH)
    pltpu.sync_copy(src_hbm.at[sl0], sidx[0])
    pltpu.sync_copy(dst_hbm.at[sl0], didx[0])
    gather_issue(0)
    idx_issue(1, 1)

    stage(0, 0, do_store_wait=False,
          do_idx=(2 <= ncht - 1), do_gather=(1 <= ncht - 1))

    npairs = max(0, (ncht - 3) // 2)

    def pair(k, carry):
        c0 = 2 * k + 1
        stage(c0, 1)
        stage(c0 + 1, 0)
        return carry

    lax.fori_loop(0, npairs, pair, 0)

    for c in range(2 * npairs + 1, ncht):
        stage(c, c % 2, do_idx=(c + 2 <= ncht - 1),
              do_gather=(c + 1 <= ncht - 1))
    store_wait((ncht - 1) % 2)


def _gather_combine(a_tab, b_tab, src, dst, eoff, n_edges, CH):
    ew = n_edges // NW
    kern = pl.kernel(
        functools.partial(_gather_body, eoff, ew, CH),
        out_type=jax.ShapeDtypeStruct((n_edges, H), jnp.float32),
        mesh=_sc_mesh(),
        scratch_types=[
            pltpu.VMEM((CH,), jnp.int32),
            pltpu.VMEM((CH,), jnp.int32),
            pltpu.VMEM((CH,), jnp.int32),
            pltpu.VMEM((CH,), jnp.int32),
            pltpu.VMEM((CH, H), jnp.float32),
            pltpu.VMEM((CH, H), jnp.float32),
            pltpu.VMEM((CH, H), jnp.float32),
            pltpu.VMEM((CH, H), jnp.float32),
            pltpu.SemaphoreType.DMA,
            pltpu.SemaphoreType.DMA,
            pltpu.SemaphoreType.DMA,
            pltpu.SemaphoreType.DMA,
            pltpu.SemaphoreType.DMA,
            pltpu.SemaphoreType.DMA,
        ],
    )
    return kern(a_tab, b_tab, src, dst)


def _edge_mlp_body(h1_ref, w2_ref, b2_ref, out_ref):
    x = jnp.maximum(h1_ref[...], 0.0)
    y = jnp.dot(x, w2_ref[...], preferred_element_type=jnp.float32)
    out_ref[...] = jnp.maximum(y + b2_ref[...], 0.0)


def _edge_mlp(h1, w2, b2):
    n_edges = h1.shape[0]
    blk = 1600
    grid = n_edges // blk
    full = lambda s: pl.BlockSpec(s, lambda i: (0,) * len(s))
    return pl.pallas_call(
        _edge_mlp_body,
        grid=(grid,),
        in_specs=[pl.BlockSpec((blk, H), lambda i: (i, 0)),
                  full((H, H)), full((1, H))],
        out_specs=pl.BlockSpec((blk, H), lambda i: (i, 0)),
        out_shape=jax.ShapeDtypeStruct((n_edges, H), jnp.float32),
    )(h1, w2, b2)


def _scatter_body(eoff, ew, CH, e2_hbm, dst_hbm, zeros_hbm, out_hbm,
                  didx0, didx1, ebuf0, ebuf1, acc, sl0_, sl1_, sc0, sc1):
    ncht = ew // CH
    cid = lax.axis_index("c")
    sid = lax.axis_index("s")
    wid = sid * NC + cid
    base_i = eoff + wid * ew
    base_o = wid * ew
    rbase = sid * RPT

    didx = (didx0, didx1)
    ebuf = (ebuf0, ebuf1)
    sl = (sl0_, sl1_)
    sc = (sc0, sc1)

    pltpu.sync_copy(zeros_hbm.at[pl.ds(rbase, RPT)],
                    acc.at[pl.ds(rbase, RPT)])

    @pl.when(sid == NS - 1)
    def _():
        pltpu.sync_copy(zeros_hbm.at[pl.ds(RPT * NS, RTAIL)],
                        acc.at[pl.ds(RPT * NS, RTAIL)])

    plsc.subcore_barrier()

    def load_issue(c, p):
        pltpu.async_copy(dst_hbm.at[pl.ds(base_i + c * CH, CH)],
                         didx[p], sl[p])
        pltpu.async_copy(e2_hbm.at[pl.ds(base_o + c * CH, CH)],
                         ebuf[p], sl[p])

    def load_wait(p):
        pltpu.make_async_copy(dst_hbm.at[pl.ds(0, CH)], didx[p], sl[p]).wait()
        pltpu.make_async_copy(e2_hbm.at[pl.ds(0, CH)], ebuf[p], sl[p]).wait()

    def scat_issue(p):
        pltpu.async_copy(ebuf[p], acc.at[didx[p]], sc[p], add=True)

    def scat_wait(p):
        pltpu.make_async_copy(ebuf[p], acc.at[didx[p]], sc[p]).wait()

    def stage(c, p, do_scat_wait=True, do_load=True):
        pn = 1 - p
        load_wait(p)
        if do_scat_wait:
            scat_wait(pn)
        if do_load:
            load_issue(c + 1, pn)
        scat_issue(p)

    load_issue(0, 0)
    stage(0, 0, do_scat_wait=False, do_load=(1 <= ncht - 1))

    npairs = max(0, (ncht - 3) // 2)

    def pair(k, carry):
        c0 = 2 * k + 1
        stage(c0, 1)
        stage(c0 + 1, 0)
        return carry

    lax.fori_loop(0, npairs, pair, 0)

    for c in range(2 * npairs + 1, ncht):
        stage(c, c % 2, do_load=(c + 1 <= ncht - 1))
    scat_wait((ncht - 1) % 2)

    plsc.subcore_barrier()
    pltpu.sync_copy(acc.at[pl.ds(rbase, RPT)],
                    out_hbm.at[pl.ds(cid * N + rbase, RPT)])

    @pl.when(sid == NS - 1)
    def _():
        pltpu.sync_copy(acc.at[pl.ds(RPT * NS, RTAIL)],
                        out_hbm.at[pl.ds(cid * N + RPT * NS, RTAIL)])


def _scatter_aggregate(e2, dst, zeros, eoff, CH):
    n_edges = e2.shape[0]
    ew = n_edges // NW
    kern = pl.kernel(
        functools.partial(_scatter_body, eoff, ew, CH),
        out_type=jax.ShapeDtypeStruct((NC * N, H), jnp.float32),
        mesh=_sc_mesh(),
        scratch_types=[
            pltpu.VMEM((CH,), jnp.int32),
            pltpu.VMEM((CH,), jnp.int32),
            pltpu.VMEM((CH, H), jnp.float32),
            pltpu.VMEM((CH, H), jnp.float32),
            pltpu.VMEM_SHARED((N, H), jnp.float32),
            pltpu.SemaphoreType.DMA,
            pltpu.SemaphoreType.DMA,
            pltpu.SemaphoreType.DMA,
            pltpu.SemaphoreType.DMA,
        ],
    )
    return kern(e2, dst, zeros)


def _decode_body(ns_ref, a0_ref, a1_ref, a2_ref, a3_ref,
                 wd1a_ref, wd1b_ref, bd1_ref, wd2_ref, bd2_ref, out_ref):
    aggr = (a0_ref[0] + a1_ref[0]) + (a2_ref[0] + a3_ref[0])
    h = jnp.dot(ns_ref[...], wd1a_ref[...],
                preferred_element_type=jnp.float32)
    h = h + jnp.dot(aggr, wd1b_ref[...],
                    preferred_element_type=jnp.float32)
    h = jnp.maximum(h + bd1_ref[...], 0.0)
    out_ref[...] = jnp.dot(h, wd2_ref[...],
                           preferred_element_type=jnp.float32) + bd2_ref[...]


def _decode(ns, aggr_a, aggr_b, wd1a, wd1b, bd1, wd2, bd2):
    blk = 1000
    grid = N // blk
    full = lambda s: pl.BlockSpec(s, lambda i: (0,) * len(s))
    part = lambda c: pl.BlockSpec((1, blk, H), lambda i, c=c: (c, i, 0))
    return pl.pallas_call(
        _decode_body,
        grid=(grid,),
        in_specs=[pl.BlockSpec((blk, D), lambda i: (i, 0)),
                  part(0), part(1), part(0), part(1),
                  full((D, H)), full((H, H)), full((1, H)),
                  full((H, D)), full((1, D))],
        out_specs=pl.BlockSpec((blk, D), lambda i: (i, 0)),
        out_shape=jax.ShapeDtypeStruct((N, D), jnp.float32),
    )(ns, aggr_a, aggr_a, aggr_b, aggr_b, wd1a, wd1b, bd1, wd2, bd2)


def kernel(node_states, edges, W_prop, b_prop, W_e1, b_e1, W_e2, b_e2,
           W_d1, b_d1, W_d2, b_d2):
    src = edges[0].astype(jnp.int32)
    dst = edges[1].astype(jnp.int32)

    a_tab, b_tab = _make_tables(
        node_states, W_prop, b_prop.reshape(1, D),
        W_e1[:D], W_e1[D:], b_e1.reshape(1, H))

    zeros = jnp.zeros((N, H), jnp.float32)
    eh2 = E - E_HALF1

    h1a = _gather_combine(a_tab, b_tab, src, dst, 0, E_HALF1, 120)
    h1b = _gather_combine(a_tab, b_tab, src, dst, E_HALF1, eh2, 128)
    e2a = _edge_mlp(h1a, W_e2, b_e2.reshape(1, H))
    e2b = _edge_mlp(h1b, W_e2, b_e2.reshape(1, H))
    aggr_a = _scatter_aggregate(e2a, dst, zeros, 0, 120).reshape(NC, N, H)
    aggr_b = _scatter_aggregate(e2b, dst, zeros, E_HALF1, 128).reshape(NC, N, H)

    return _decode(node_states, aggr_a, aggr_b, W_d1[:D], W_d1[D:],
                   b_d1.reshape(1, H), W_d2, b_d2.reshape(1, D))


# speedup vs baseline: 1.0196x; 1.0196x over previous
import functools

import jax
import jax.numpy as jnp
from jax import lax
from jax.experimental import pallas as pl
from jax.experimental.pallas import tpu as pltpu
from jax.experimental.pallas import tpu_sc as plsc

N = 10000
E = 320000
D = 128
H = 128

NC = 2
NS = 16
LN = 16
NW = NC * NS
E_HALF1 = 128000
RPT = 624
RTAIL = N - RPT * NS

_sc_mesh = functools.partial(
    plsc.VectorSubcoreMesh,
    core_axis_name="c", subcore_axis_name="s", num_cores=NC, num_subcores=NS)


def _tables_body(ns_ref, wp_ref, bp_ref, w1a_ref, w1b_ref, be1_ref,
                 a_ref, b_ref):
    nm = jnp.dot(ns_ref[...], wp_ref[...],
                 preferred_element_type=jnp.float32) + bp_ref[...]
    a_ref[...] = jnp.dot(nm, w1a_ref[...],
                         preferred_element_type=jnp.float32) + be1_ref[...]
    b_ref[...] = jnp.dot(nm, w1b_ref[...],
                         preferred_element_type=jnp.float32)


def _make_tables(ns, wp, bp, w1a, w1b, be1):
    blk = 1000
    grid = N // blk
    row_spec = pl.BlockSpec((blk, D), lambda i: (i, 0))
    full = lambda s: pl.BlockSpec(s, lambda i: (0,) * len(s))
    return pl.pallas_call(
        _tables_body,
        grid=(grid,),
        in_specs=[row_spec, full((D, D)), full((1, D)),
                  full((D, H)), full((D, H)), full((1, H))],
        out_specs=[pl.BlockSpec((blk, H), lambda i: (i, 0))] * 2,
        out_shape=[jax.ShapeDtypeStruct((N, H), jnp.float32)] * 2,
    )(ns, wp, bp, w1a, w1b, be1)


def _gather_body(eoff, ew, CH, a_hbm, b_hbm, src_hbm, dst_hbm, h1_hbm,
                 sidx0, sidx1, didx0, didx1, abuf0, abuf1, bbuf0, bbuf1,
                 gi0, gi1, gg0, gg1, gs0, gs1):
    ncht = ew // CH
    cid = lax.axis_index("c")
    sid = lax.axis_index("s")
    wid = sid * NC + cid
    base_i = eoff + wid * ew
    base_o = wid * ew

    sidx = (sidx0, sidx1)
    didx = (didx0, didx1)
    abuf = (abuf0, abuf1)
    bbuf = (bbuf0, bbuf1)
    gi = (gi0, gi1)
    gg = (gg0, gg1)
    gs = (gs0, gs1)

    def idx_issue(c, p):
        sl = pl.ds(base_i + c * CH, CH)
        pltpu.async_copy(src_hbm.at[sl], sidx[p], gi[p])
        pltpu.async_copy(dst_hbm.at[sl], didx[p], gi[p])

    def idx_wait(p):
        pltpu.make_async_copy(src_hbm.at[pl.ds(0, CH)], sidx[p], gi[p]).wait()
        pltpu.make_async_copy(dst_hbm.at[pl.ds(0, CH)], didx[p], gi[p]).wait()

    def gather_issue(p):
        pltpu.async_copy(a_hbm.at[sidx[p]], abuf[p], gg[p])
        pltpu.async_copy(b_hbm.at[didx[p]], bbuf[p], gg[p])

    def gather_wait(p):
        pltpu.make_async_copy(a_hbm.at[sidx[p]], abuf[p], gg[p]).wait()
        pltpu.make_async_copy(b_hbm.at[didx[p]], bbuf[p], gg[p]).wait()

    def store_wait(p):
        pltpu.make_async_copy(
            abuf[p], h1_hbm.at[pl.ds(base_o, CH)], gs[p]).wait()

    def stage(c, p, do_store_wait=True, do_idx=True, do_gather=True):
        pn = 1 - p
        gather_wait(p)
        if do_idx:
            idx_issue(c + 2, p)
        if do_store_wait:
            store_wait(pn)
        if do_gather:
            idx_wait(pn)
            gather_issue(pn)

        def row(r, c2):
            for j in range(H // LN):
                sl = pl.ds(j * LN, LN)
                abuf[p][r, sl] = abuf[p][r, sl] + bbuf[p][r, sl]
            return c2

        lax.fori_loop(0, CH, row, 0)
        pltpu.async_copy(abuf[p], h1_hbm.at[pl.ds(base_o + c * CH, CH)],
                         gs[p])

    sl0 = pl.ds(base_i, CH)
    pltpu.sync_copy(src_hbm.at[sl0], sidx[0])
    pltpu.sync_copy(dst_hbm.at[sl0], didx[0])
    gather_issue(0)
    idx_issue(1, 1)

    stage(0, 0, do_store_wait=False,
          do_idx=(2 <= ncht - 1), do_gather=(1 <= ncht - 1))

    npairs = max(0, (ncht - 3) // 2)

    def pair(k, carry):
        c0 = 2 * k + 1
        stage(c0, 1)
        stage(c0 + 1, 0)
        return carry

    lax.fori_loop(0, npairs, pair, 0)

    for c in range(2 * npairs + 1, ncht):
        stage(c, c % 2, do_idx=(c + 2 <= ncht - 1),
              do_gather=(c + 1 <= ncht - 1))
    store_wait((ncht - 1) % 2)


def _gather_combine(a_tab, b_tab, src, dst, eoff, n_edges, CH):
    ew = n_edges // NW
    kern = pl.kernel(
        functools.partial(_gather_body, eoff, ew, CH),
        out_type=jax.ShapeDtypeStruct((n_edges, H), jnp.float32),
        mesh=_sc_mesh(),
        scratch_types=[
            pltpu.VMEM((CH,), jnp.int32),
            pltpu.VMEM((CH,), jnp.int32),
            pltpu.VMEM((CH,), jnp.int32),
            pltpu.VMEM((CH,), jnp.int32),
            pltpu.VMEM((CH, H), jnp.float32),
            pltpu.VMEM((CH, H), jnp.float32),
            pltpu.VMEM((CH, H), jnp.float32),
            pltpu.VMEM((CH, H), jnp.float32),
            pltpu.SemaphoreType.DMA,
            pltpu.SemaphoreType.DMA,
            pltpu.SemaphoreType.DMA,
            pltpu.SemaphoreType.DMA,
            pltpu.SemaphoreType.DMA,
            pltpu.SemaphoreType.DMA,
        ],
    )
    return kern(a_tab, b_tab, src, dst)


def _edge_mlp_body(h1_ref, w2_ref, b2_ref, out_ref):
    x = jnp.maximum(h1_ref[...], 0.0)
    y = jnp.dot(x, w2_ref[...], preferred_element_type=jnp.float32)
    out_ref[...] = jnp.maximum(y + b2_ref[...], 0.0)


def _edge_mlp(h1, w2, b2):
    n_edges = h1.shape[0]
    blk = 2000
    grid = n_edges // blk
    full = lambda s: pl.BlockSpec(s, lambda i: (0,) * len(s))
    return pl.pallas_call(
        _edge_mlp_body,
        grid=(grid,),
        in_specs=[pl.BlockSpec((blk, H), lambda i: (i, 0)),
                  full((H, H)), full((1, H))],
        out_specs=pl.BlockSpec((blk, H), lambda i: (i, 0)),
        out_shape=jax.ShapeDtypeStruct((n_edges, H), jnp.float32),
    )(h1, w2, b2)


def _scatter_body(eoff, ew, CH, e2_hbm, dst_hbm, zeros_hbm, out_hbm,
                  didx0, didx1, ebuf0, ebuf1, acc, sl0_, sl1_, sc0, sc1):
    ncht = ew // CH
    cid = lax.axis_index("c")
    sid = lax.axis_index("s")
    wid = sid * NC + cid
    base_i = eoff + wid * ew
    base_o = wid * ew
    rbase = sid * RPT

    didx = (didx0, didx1)
    ebuf = (ebuf0, ebuf1)
    sl = (sl0_, sl1_)
    sc = (sc0, sc1)

    pltpu.sync_copy(zeros_hbm.at[pl.ds(rbase, RPT)],
                    acc.at[pl.ds(rbase, RPT)])

    @pl.when(sid == NS - 1)
    def _():
        pltpu.sync_copy(zeros_hbm.at[pl.ds(RPT * NS, RTAIL)],
                        acc.at[pl.ds(RPT * NS, RTAIL)])

    plsc.subcore_barrier()

    def load_issue(c, p):
        pltpu.async_copy(dst_hbm.at[pl.ds(base_i + c * CH, CH)],
                         didx[p], sl[p])
        pltpu.async_copy(e2_hbm.at[pl.ds(base_o + c * CH, CH)],
                         ebuf[p], sl[p])

    def load_wait(p):
        pltpu.make_async_copy(dst_hbm.at[pl.ds(0, CH)], didx[p], sl[p]).wait()
        pltpu.make_async_copy(e2_hbm.at[pl.ds(0, CH)], ebuf[p], sl[p]).wait()

    def scat_issue(p):
        pltpu.async_copy(ebuf[p], acc.at[didx[p]], sc[p], add=True)

    def scat_wait(p):
        pltpu.make_async_copy(ebuf[p], acc.at[didx[p]], sc[p]).wait()

    def stage(c, p, do_scat_wait=True, do_load=True):
        pn = 1 - p
        load_wait(p)
        if do_scat_wait:
            scat_wait(pn)
        if do_load:
            load_issue(c + 1, pn)
        scat_issue(p)

    load_issue(0, 0)
    stage(0, 0, do_scat_wait=False, do_load=(1 <= ncht - 1))

    npairs = max(0, (ncht - 3) // 2)

    def pair(k, carry):
        c0 = 2 * k + 1
        stage(c0, 1)
        stage(c0 + 1, 0)
        return carry

    lax.fori_loop(0, npairs, pair, 0)

    for c in range(2 * npairs + 1, ncht):
        stage(c, c % 2, do_load=(c + 1 <= ncht - 1))
    scat_wait((ncht - 1) % 2)

    plsc.subcore_barrier()
    pltpu.sync_copy(acc.at[pl.ds(rbase, RPT)],
                    out_hbm.at[pl.ds(cid * N + rbase, RPT)])

    @pl.when(sid == NS - 1)
    def _():
        pltpu.sync_copy(acc.at[pl.ds(RPT * NS, RTAIL)],
                        out_hbm.at[pl.ds(cid * N + RPT * NS, RTAIL)])


def _scatter_aggregate(e2, dst, zeros, eoff, CH):
    n_edges = e2.shape[0]
    ew = n_edges // NW
    kern = pl.kernel(
        functools.partial(_scatter_body, eoff, ew, CH),
        out_type=jax.ShapeDtypeStruct((NC * N, H), jnp.float32),
        mesh=_sc_mesh(),
        scratch_types=[
            pltpu.VMEM((CH,), jnp.int32),
            pltpu.VMEM((CH,), jnp.int32),
            pltpu.VMEM((CH, H), jnp.float32),
            pltpu.VMEM((CH, H), jnp.float32),
            pltpu.VMEM_SHARED((N, H), jnp.float32),
            pltpu.SemaphoreType.DMA,
            pltpu.SemaphoreType.DMA,
            pltpu.SemaphoreType.DMA,
            pltpu.SemaphoreType.DMA,
        ],
    )
    return kern(e2, dst, zeros)


def _decode_body(ns_ref, a0_ref, a1_ref, a2_ref, a3_ref,
                 wd1a_ref, wd1b_ref, bd1_ref, wd2_ref, bd2_ref, out_ref):
    aggr = (a0_ref[0] + a1_ref[0]) + (a2_ref[0] + a3_ref[0])
    h = jnp.dot(ns_ref[...], wd1a_ref[...],
                preferred_element_type=jnp.float32)
    h = h + jnp.dot(aggr, wd1b_ref[...],
                    preferred_element_type=jnp.float32)
    h = jnp.maximum(h + bd1_ref[...], 0.0)
    out_ref[...] = jnp.dot(h, wd2_ref[...],
                           preferred_element_type=jnp.float32) + bd2_ref[...]


def _decode(ns, aggr_a, aggr_b, wd1a, wd1b, bd1, wd2, bd2):
    blk = 1000
    grid = N // blk
    full = lambda s: pl.BlockSpec(s, lambda i: (0,) * len(s))
    part = lambda c: pl.BlockSpec((1, blk, H), lambda i, c=c: (c, i, 0))
    return pl.pallas_call(
        _decode_body,
        grid=(grid,),
        in_specs=[pl.BlockSpec((blk, D), lambda i: (i, 0)),
                  part(0), part(1), part(0), part(1),
                  full((D, H)), full((H, H)), full((1, H)),
                  full((H, D)), full((1, D))],
        out_specs=pl.BlockSpec((blk, D), lambda i: (i, 0)),
        out_shape=jax.ShapeDtypeStruct((N, D), jnp.float32),
    )(ns, aggr_a, aggr_a, aggr_b, aggr_b, wd1a, wd1b, bd1, wd2, bd2)


def kernel(node_states, edges, W_prop, b_prop, W_e1, b_e1, W_e2, b_e2,
           W_d1, b_d1, W_d2, b_d2):
    src = edges[0].astype(jnp.int32)
    dst = edges[1].astype(jnp.int32)

    a_tab, b_tab = _make_tables(
        node_states, W_prop, b_prop.reshape(1, D),
        W_e1[:D], W_e1[D:], b_e1.reshape(1, H))

    zeros = jnp.zeros((N, H), jnp.float32)
    eh2 = E - E_HALF1

    h1a = _gather_combine(a_tab, b_tab, src, dst, 0, E_HALF1, 80)
    h1b = _gather_combine(a_tab, b_tab, src, dst, E_HALF1, eh2, 120)
    e2a = _edge_mlp(h1a, W_e2, b_e2.reshape(1, H))
    e2b = _edge_mlp(h1b, W_e2, b_e2.reshape(1, H))
    aggr_a = _scatter_aggregate(e2a, dst, zeros, 0, 80).reshape(NC, N, H)
    aggr_b = _scatter_aggregate(e2b, dst, zeros, E_HALF1, 120).reshape(NC, N, H)

    return _decode(node_states, aggr_a, aggr_b, W_d1[:D], W_d1[D:],
                   b_d1.reshape(1, H), W_d2, b_d2.reshape(1, D))
